# baseline (speedup 1.0000x reference)
import functools

import jax
import jax.numpy as jnp
from jax import lax
from jax.experimental import pallas as pl
from jax.experimental.pallas import tpu as pltpu
from jax.experimental.pallas import tpu_sc as plsc

_LANES = 16
_CHUNK = 128


def _make_loss_kernel(num_workers, nc, b_per_w, k_dim):
    n_chunks = b_per_w // _CHUNK
    n_groups = b_per_w // _LANES
    mesh = plsc.VectorSubcoreMesh(core_axis_name="c", subcore_axis_name="s")

    @functools.partial(
        pl.kernel,
        mesh=mesh,
        out_type=jax.ShapeDtypeStruct((num_workers, _LANES), jnp.float32),
        compiler_params=pltpu.CompilerParams(
            needs_layout_passes=False, use_tc_tiling_on_sc=False),
        scratch_types=[
            pltpu.VMEM((n_chunks, _CHUNK), jnp.int32),
            pltpu.VMEM((n_chunks, _CHUNK), jnp.int32),
            pltpu.VMEM((b_per_w,), jnp.float32),
            pltpu.VMEM((_LANES,), jnp.float32),
            pltpu.VMEM((b_per_w,), jnp.float32),
            pltpu.VMEM((b_per_w,), jnp.float32),
            pltpu.VMEM((k_dim, b_per_w), jnp.float32),
            pltpu.VMEM((k_dim, b_per_w), jnp.float32),
            pltpu.VMEM((_LANES,), jnp.float32),
            pltpu.SemaphoreType.DMA,
            pltpu.SemaphoreType.DMA,
        ],
    )
    def loss_kernel(su_hbm, si_hbm, r_hbm, av_hbm, bU_hbm, bI_hbm, gUT_hbm,
                    gIT_hbm, out_hbm, idx_u, idx_i, r_v, a_v, bu_v, bi_v,
                    gu_v, gi_v, loss_v, sem, gsem):
        wid = lax.axis_index("s") * nc + lax.axis_index("c")
        pltpu.sync_copy(su_hbm.at[wid], idx_u)
        pltpu.sync_copy(si_hbm.at[wid], idx_i)
        pltpu.sync_copy(r_hbm.at[wid], r_v)
        pltpu.sync_copy(av_hbm, a_v)
        copies = []
        for j in range(n_chunks):
            sl = pl.ds(j * _CHUNK, _CHUNK)
            copies.append(
                pltpu.async_copy(bU_hbm.at[idx_u.at[j]], bu_v.at[sl], sem))
            copies.append(
                pltpu.async_copy(bI_hbm.at[idx_i.at[j]], bi_v.at[sl], sem))

        for j in range(n_chunks):
            sl = pl.ds(j * _CHUNK, _CHUNK)
            for k in range(k_dim):
                copies.append(pltpu.async_copy(
                    gUT_hbm.at[k].at[idx_u.at[j]], gu_v.at[k].at[sl], gsem))
                copies.append(pltpu.async_copy(
                    gIT_hbm.at[k].at[idx_i.at[j]], gi_v.at[k].at[sl], gsem))
        for c in copies:
            c.wait()
        alpha = a_v[...]

        def group(g, acc_loss):
            sl = pl.ds(g * _LANES, _LANES)
            dot = gu_v[0, sl] * gi_v[0, sl]
            for k in range(1, k_dim):
                dot = dot + gu_v[k, sl] * gi_v[k, sl]
            pred = alpha + bu_v[sl] + bi_v[sl] + dot
            sig = 5.0 / (1.0 + jnp.exp(-pred))
            diff = sig - r_v[sl]
            return acc_loss + diff * diff

        acc = lax.fori_loop(0, n_groups, group,
                            jnp.zeros((_LANES,), jnp.float32))
        loss_v[...] = acc
        pltpu.sync_copy(loss_v, out_hbm.at[wid])

    return loss_kernel


def kernel(sampleU, sampleI, sampleR, alpha, betaU, betaI, gammaU, gammaI):
    info = plsc.get_sparse_core_info()
    nc, ns = info.num_cores, info.num_subcores
    nw = nc * ns
    b = sampleU.shape[0]
    k_dim = gammaU.shape[1]
    b_per_w = b // nw
    su = sampleU.astype(jnp.int32).reshape(nw, b_per_w // _CHUNK, _CHUNK)
    si = sampleI.astype(jnp.int32).reshape(nw, b_per_w // _CHUNK, _CHUNK)
    r = sampleR.astype(jnp.float32).reshape(nw, b_per_w)
    av = jnp.broadcast_to(jnp.asarray(alpha, jnp.float32), (_LANES,))
    fn = _make_loss_kernel(nw, nc, b_per_w, k_dim)
    out = fn(su, si, r, av, betaU, betaI, gammaU.T, gammaI.T)
    return jnp.sum(out) / b


# speedup vs baseline: 17.1546x; 17.1546x over previous
import functools

import jax
import jax.numpy as jnp
from jax import lax
from jax.experimental import pallas as pl
from jax.experimental.pallas import tpu as pltpu
from jax.experimental.pallas import tpu_sc as plsc

_LANES = 16
_CHUNK = 128
_SUB = 8


def _flatten_table(table):
    v, k = table.shape
    padc = (-v) % _CHUNK
    nt = (v + padc) // _CHUNK
    tp = jnp.pad(table.T, ((0, 0), (0, padc)))
    flat = tp.reshape(k // _SUB, _SUB, nt, _CHUNK)
    flat = flat.transpose(0, 2, 1, 3).reshape(-1)
    return flat, nt


def _make_loss_kernel(num_workers, nc, b_per_w, k_dim, nt_u, nt_i):
    n_chunks = b_per_w // _CHUNK
    n_groups = b_per_w // _LANES
    mesh = plsc.VectorSubcoreMesh(core_axis_name="c", subcore_axis_name="s")

    @functools.partial(
        pl.kernel,
        mesh=mesh,
        out_type=jax.ShapeDtypeStruct((num_workers, _LANES), jnp.float32),
        compiler_params=pltpu.CompilerParams(
            needs_layout_passes=False, use_tc_tiling_on_sc=False),
        scratch_types=[
            pltpu.VMEM((n_chunks, _CHUNK), jnp.int32),
            pltpu.VMEM((n_chunks, _CHUNK), jnp.int32),
            pltpu.VMEM((b_per_w,), jnp.float32),
            pltpu.VMEM((_LANES,), jnp.float32),
            pltpu.VMEM((b_per_w,), jnp.float32),
            pltpu.VMEM((b_per_w,), jnp.float32),
            pltpu.VMEM((k_dim, n_chunks, _CHUNK), jnp.int32),
            pltpu.VMEM((k_dim, n_chunks, _CHUNK), jnp.int32),
            pltpu.VMEM((k_dim, b_per_w), jnp.float32),
            pltpu.VMEM((k_dim, b_per_w), jnp.float32),
            pltpu.VMEM((_LANES,), jnp.float32),
            pltpu.SemaphoreType.DMA,
            pltpu.SemaphoreType.DMA,
        ],
    )
    def loss_kernel(su_hbm, si_hbm, r_hbm, av_hbm, bU_hbm, bI_hbm, gU_hbm,
                    gI_hbm, out_hbm, idx_u, idx_i, r_v, a_v, bu_v, bi_v,
                    gidx_u, gidx_i, gu_v, gi_v, loss_v, sem, gsem):
        wid = lax.axis_index("s") * nc + lax.axis_index("c")
        pltpu.sync_copy(su_hbm.at[wid], idx_u)
        pltpu.sync_copy(si_hbm.at[wid], idx_i)
        pltpu.sync_copy(r_hbm.at[wid], r_v)
        pltpu.sync_copy(av_hbm, a_v)
        beta_copies = []
        for j in range(n_chunks):
            sl = pl.ds(j * _CHUNK, _CHUNK)
            beta_copies.append(
                pltpu.async_copy(bU_hbm.at[idx_u.at[j]], bu_v.at[sl], sem))
            beta_copies.append(
                pltpu.async_copy(bI_hbm.at[idx_i.at[j]], bi_v.at[sl], sem))

        def calc_idx(jt, carry):
            j = jt // (_CHUNK // _LANES)
            t = jt % (_CHUNK // _LANES)
            tsl = pl.ds(t * _LANES, _LANES)
            uvec = idx_u[j, tsl]
            ivec = idx_i[j, tsl]
            ubase = ((uvec >> 7) << 10) + (uvec & 127)
            ibase = ((ivec >> 7) << 10) + (ivec & 127)
            for k in range(k_dim):
                ku = ((k // _SUB) * nt_u << 10) + (k % _SUB) * _CHUNK
                ki = ((k // _SUB) * nt_i << 10) + (k % _SUB) * _CHUNK
                gidx_u[k, j, tsl] = ubase + ku
                gidx_i[k, j, tsl] = ibase + ki
            return carry

        lax.fori_loop(0, b_per_w // _LANES, calc_idx, 0)

        def fire(k, carry):
            for j in range(n_chunks):
                sl = pl.ds(j * _CHUNK, _CHUNK)
                pltpu.async_copy(gU_hbm.at[gidx_u.at[k].at[j]],
                                 gu_v.at[k].at[sl], gsem)
                pltpu.async_copy(gI_hbm.at[gidx_i.at[k].at[j]],
                                 gi_v.at[k].at[sl], gsem)
            return carry

        lax.fori_loop(0, k_dim, fire, 0)

        def drain(d, carry):
            pltpu.make_async_copy(gU_hbm.at[pl.ds(0, _CHUNK)],
                                  gu_v.at[0].at[pl.ds(0, _CHUNK)],
                                  gsem).wait()
            pltpu.make_async_copy(gI_hbm.at[pl.ds(0, _CHUNK)],
                                  gi_v.at[0].at[pl.ds(0, _CHUNK)],
                                  gsem).wait()
            return carry

        lax.fori_loop(0, k_dim * n_chunks, drain, 0)
        for c in beta_copies:
            c.wait()
        alpha = a_v[...]

        def group(g, acc_loss):
            sl = pl.ds(g * _LANES, _LANES)
            dot = gu_v[0, sl] * gi_v[0, sl]
            for k in range(1, k_dim):
                dot = dot + gu_v[k, sl] * gi_v[k, sl]
            pred = alpha + bu_v[sl] + bi_v[sl] + dot
            sig = 5.0 / (1.0 + jnp.exp(-pred))
            diff = sig - r_v[sl]
            return acc_loss + diff * diff

        acc = lax.fori_loop(0, n_groups, group,
                            jnp.zeros((_LANES,), jnp.float32))
        loss_v[...] = acc
        pltpu.sync_copy(loss_v, out_hbm.at[wid])

    return loss_kernel


def kernel(sampleU, sampleI, sampleR, alpha, betaU, betaI, gammaU, gammaI):
    info = plsc.get_sparse_core_info()
    nc, ns = info.num_cores, info.num_subcores
    nw = nc * ns
    b = sampleU.shape[0]
    k_dim = gammaU.shape[1]
    b_per_w = b // nw
    su = sampleU.astype(jnp.int32).reshape(nw, b_per_w // _CHUNK, _CHUNK)
    si = sampleI.astype(jnp.int32).reshape(nw, b_per_w // _CHUNK, _CHUNK)
    r = sampleR.astype(jnp.float32).reshape(nw, b_per_w)
    av = jnp.broadcast_to(jnp.asarray(alpha, jnp.float32), (_LANES,))
    gU_flat, nt_u = _flatten_table(gammaU)
    gI_flat, nt_i = _flatten_table(gammaI)
    fn = _make_loss_kernel(nw, nc, b_per_w, k_dim, nt_u, nt_i)
    out = fn(su, si, r, av, betaU, betaI, gU_flat, gI_flat)
    return jnp.sum(out) / b


# speedup vs baseline: 17.1588x; 1.0002x over previous
import functools

import jax
import jax.numpy as jnp
from jax import lax
from jax.experimental import pallas as pl
from jax.experimental.pallas import tpu as pltpu
from jax.experimental.pallas import tpu_sc as plsc

_LANES = 16
_BLK = 128
_SUB = 8


def _flatten_table(table):
    v, k = table.shape
    padc = (-v) % _BLK
    nt = (v + padc) // _BLK
    tp = jnp.pad(table.T, ((0, 0), (0, padc)))
    flat = tp.reshape(k // _SUB, _SUB, nt, _BLK)
    flat = flat.transpose(0, 2, 1, 3).reshape(-1)
    return flat, nt


def _make_loss_kernel(num_workers, nc, b_per_w, k_dim, nt_u, nt_i):
    n_groups = b_per_w // _LANES
    gather_bytes = 2 * k_dim * b_per_w * 4
    mesh = plsc.VectorSubcoreMesh(core_axis_name="c", subcore_axis_name="s")

    @functools.partial(
        pl.kernel,
        mesh=mesh,
        out_type=jax.ShapeDtypeStruct((num_workers, _LANES), jnp.float32),
        compiler_params=pltpu.CompilerParams(
            needs_layout_passes=False, use_tc_tiling_on_sc=False),
        scratch_types=[
            pltpu.VMEM((b_per_w,), jnp.int32),
            pltpu.VMEM((b_per_w,), jnp.int32),
            pltpu.VMEM((b_per_w,), jnp.float32),
            pltpu.VMEM((_LANES,), jnp.float32),
            pltpu.VMEM((b_per_w,), jnp.float32),
            pltpu.VMEM((b_per_w,), jnp.float32),
            pltpu.VMEM((k_dim, b_per_w), jnp.int32),
            pltpu.VMEM((k_dim, b_per_w), jnp.int32),
            pltpu.VMEM((k_dim, b_per_w), jnp.float32),
            pltpu.VMEM((k_dim, b_per_w), jnp.float32),
            pltpu.VMEM((_LANES,), jnp.float32),
            pltpu.SemaphoreType.DMA,
            pltpu.SemaphoreType.DMA,
        ],
    )
    def loss_kernel(su_hbm, si_hbm, r_hbm, av_hbm, bU_hbm, bI_hbm, gU_hbm,
                    gI_hbm, out_hbm, idx_u, idx_i, r_v, a_v, bu_v, bi_v,
                    gidx_u, gidx_i, gu_v, gi_v, loss_v, sem, gsem):
        wid = lax.axis_index("s") * nc + lax.axis_index("c")
        pltpu.sync_copy(su_hbm.at[wid], idx_u)
        pltpu.sync_copy(si_hbm.at[wid], idx_i)
        pltpu.sync_copy(r_hbm.at[wid], r_v)
        pltpu.sync_copy(av_hbm, a_v)
        bu_copy = pltpu.async_copy(bU_hbm.at[idx_u], bu_v, sem)
        bi_copy = pltpu.async_copy(bI_hbm.at[idx_i], bi_v, sem)

        def calc_idx(t, carry):
            tsl = pl.ds(t * _LANES, _LANES)
            uvec = idx_u[tsl]
            ivec = idx_i[tsl]
            ubase = ((uvec >> 7) << 10) + (uvec & 127)
            ibase = ((ivec >> 7) << 10) + (ivec & 127)
            for k in range(k_dim):
                ku = ((k // _SUB) * nt_u << 10) + (k % _SUB) * _BLK
                ki = ((k // _SUB) * nt_i << 10) + (k % _SUB) * _BLK
                gidx_u[k, tsl] = ubase + ku
                gidx_i[k, tsl] = ibase + ki
            return carry

        lax.fori_loop(0, b_per_w // _LANES, calc_idx, 0)

        def fire(k, carry):
            pltpu.async_copy(gU_hbm.at[gidx_u.at[k]], gu_v.at[k], gsem)
            pltpu.async_copy(gI_hbm.at[gidx_i.at[k]], gi_v.at[k], gsem)
            return carry

        lax.fori_loop(0, k_dim, fire, 0)

        def drain(d, carry):
            pltpu.make_async_copy(gU_hbm.at[pl.ds(0, b_per_w)],
                                  gu_v.at[0], gsem).wait()
            pltpu.make_async_copy(gI_hbm.at[pl.ds(0, b_per_w)],
                                  gi_v.at[0], gsem).wait()
            return carry

        lax.fori_loop(0, k_dim, drain, 0)
        bu_copy.wait()
        bi_copy.wait()
        alpha = a_v[...]

        def group(g, acc_loss):
            sl = pl.ds(g * _LANES, _LANES)
            dot = gu_v[0, sl] * gi_v[0, sl]
            for k in range(1, k_dim):
                dot = dot + gu_v[k, sl] * gi_v[k, sl]
            pred = alpha + bu_v[sl] + bi_v[sl] + dot
            sig = 5.0 / (1.0 + jnp.exp(-pred))
            diff = sig - r_v[sl]
            return acc_loss + diff * diff

        acc = lax.fori_loop(0, n_groups, group,
                            jnp.zeros((_LANES,), jnp.float32))
        loss_v[...] = acc
        pltpu.sync_copy(loss_v, out_hbm.at[wid])

    return loss_kernel


def kernel(sampleU, sampleI, sampleR, alpha, betaU, betaI, gammaU, gammaI):
    info = plsc.get_sparse_core_info()
    nc, ns = info.num_cores, info.num_subcores
    nw = nc * ns
    b = sampleU.shape[0]
    k_dim = gammaU.shape[1]
    b_per_w = b // nw
    su = sampleU.astype(jnp.int32).reshape(nw, b_per_w)
    si = sampleI.astype(jnp.int32).reshape(nw, b_per_w)
    r = sampleR.astype(jnp.float32).reshape(nw, b_per_w)
    av = jnp.broadcast_to(jnp.asarray(alpha, jnp.float32), (_LANES,))
    gU_flat, nt_u = _flatten_table(gammaU)
    gI_flat, nt_i = _flatten_table(gammaI)
    fn = _make_loss_kernel(nw, nc, b_per_w, k_dim, nt_u, nt_i)
    out = fn(su, si, r, av, betaU, betaI, gU_flat, gI_flat)
    return jnp.sum(out) / b


# speedup vs baseline: 18.1598x; 1.0583x over previous
import functools

import jax
import jax.numpy as jnp
from jax import lax
from jax.experimental import pallas as pl
from jax.experimental.pallas import tpu as pltpu
from jax.experimental.pallas import tpu_sc as plsc

_LANES = 16
_BLK = 128
_SUB = 8


def _flatten_table(table):
    v, k = table.shape
    padc = (-v) % _BLK
    nt = (v + padc) // _BLK
    tp = jnp.pad(table.T, ((0, 0), (0, padc)))
    flat = tp.reshape(k // _SUB, _SUB, nt, _BLK)
    flat = flat.transpose(0, 2, 1, 3).reshape(-1)
    return flat, nt


def _flat_indices(ids, nt, k, gidx):
    def calc_idx(t, carry):
        tsl = pl.ds(t * _LANES, _LANES)
        vec = ids[tsl]
        base = ((vec >> 7) << 10) + (vec & 127)
        for kk in range(k):
            off = ((kk // _SUB) * nt << 10) + (kk % _SUB) * _BLK
            gidx[kk, tsl] = base + off
        return carry

    lax.fori_loop(0, ids.shape[0] // _LANES, calc_idx, 0)


def _mesh():
    return plsc.VectorSubcoreMesh(core_axis_name="c", subcore_axis_name="s")


_PARAMS = None


def _compiler_params():
    return pltpu.CompilerParams(
        needs_layout_passes=False, use_tc_tiling_on_sc=False)


def _make_prep_kernel(num_workers, nc, b_per_w, k_dim, nt_i):
    @functools.partial(
        pl.kernel,
        mesh=_mesh(),
        out_type=(
            jax.ShapeDtypeStruct((num_workers, b_per_w), jnp.float32),
            jax.ShapeDtypeStruct((num_workers, b_per_w), jnp.float32),
            jax.ShapeDtypeStruct((num_workers, k_dim, b_per_w), jnp.float32),
        ),
        compiler_params=_compiler_params(),
        scratch_types=[
            pltpu.VMEM((b_per_w,), jnp.int32),
            pltpu.VMEM((b_per_w,), jnp.int32),
            pltpu.VMEM((b_per_w,), jnp.float32),
            pltpu.VMEM((b_per_w,), jnp.float32),
            pltpu.VMEM((k_dim, b_per_w), jnp.int32),
            pltpu.VMEM((k_dim, b_per_w), jnp.float32),
            pltpu.SemaphoreType.DMA,
            pltpu.SemaphoreType.DMA,
        ],
    )
    def prep_kernel(su_hbm, si_hbm, bU_hbm, bI_hbm, gI_hbm, bu_out, bi_out,
                    giv_out, idx_u, idx_i, bu_v, bi_v, gidx_i, gi_v, sem,
                    gsem):
        wid = lax.axis_index("s") * nc + lax.axis_index("c")
        pltpu.sync_copy(su_hbm.at[wid], idx_u)
        pltpu.sync_copy(si_hbm.at[wid], idx_i)
        bu_copy = pltpu.async_copy(bU_hbm.at[idx_u], bu_v, sem)
        bi_copy = pltpu.async_copy(bI_hbm.at[idx_i], bi_v, sem)
        _flat_indices(idx_i, nt_i, k_dim, gidx_i)

        def fire(k, carry):
            pltpu.async_copy(gI_hbm.at[gidx_i.at[k]], gi_v.at[k], gsem)
            return carry

        lax.fori_loop(0, k_dim, fire, 0)

        def drain(d, carry):
            pltpu.make_async_copy(gI_hbm.at[pl.ds(0, b_per_w)],
                                  gi_v.at[0], gsem).wait()
            return carry

        lax.fori_loop(0, k_dim, drain, 0)
        bu_copy.wait()
        bi_copy.wait()
        pltpu.sync_copy(bu_v, bu_out.at[wid])
        pltpu.sync_copy(bi_v, bi_out.at[wid])
        pltpu.sync_copy(gi_v, giv_out.at[wid])

    return prep_kernel


def _make_main_kernel(num_workers, nc, b_per_w, k_dim, nt_u):
    n_groups = b_per_w // _LANES

    @functools.partial(
        pl.kernel,
        mesh=_mesh(),
        out_type=jax.ShapeDtypeStruct((num_workers, _LANES), jnp.float32),
        compiler_params=_compiler_params(),
        scratch_types=[
            pltpu.VMEM((b_per_w,), jnp.int32),
            pltpu.VMEM((b_per_w,), jnp.float32),
            pltpu.VMEM((_LANES,), jnp.float32),
            pltpu.VMEM((b_per_w,), jnp.float32),
            pltpu.VMEM((b_per_w,), jnp.float32),
            pltpu.VMEM((k_dim, b_per_w), jnp.int32),
            pltpu.VMEM((k_dim, b_per_w), jnp.float32),
            pltpu.VMEM((k_dim, b_per_w), jnp.float32),
            pltpu.VMEM((_LANES,), jnp.float32),
            pltpu.SemaphoreType.DMA,
        ],
    )
    def main_kernel(su_hbm, r_hbm, av_hbm, gU_hbm, bu_hbm, bi_hbm, giv_hbm,
                    out_hbm, idx_u, r_v, a_v, bu_v, bi_v, gidx_u, gu_v, gi_v,
                    loss_v, gsem):
        wid = lax.axis_index("s") * nc + lax.axis_index("c")
        pltpu.sync_copy(su_hbm.at[wid], idx_u)
        _flat_indices(idx_u, nt_u, k_dim, gidx_u)

        def fire(k, carry):
            pltpu.async_copy(gU_hbm.at[gidx_u.at[k]], gu_v.at[k], gsem)
            return carry

        lax.fori_loop(0, k_dim, fire, 0)
        pltpu.sync_copy(r_hbm.at[wid], r_v)
        pltpu.sync_copy(av_hbm, a_v)
        pltpu.sync_copy(bu_hbm.at[wid], bu_v)
        pltpu.sync_copy(bi_hbm.at[wid], bi_v)
        pltpu.sync_copy(giv_hbm.at[wid], gi_v)

        def drain(d, carry):
            pltpu.make_async_copy(gU_hbm.at[pl.ds(0, b_per_w)],
                                  gu_v.at[0], gsem).wait()
            return carry

        lax.fori_loop(0, k_dim, drain, 0)
        alpha = a_v[...]

        def group(g, acc_loss):
            sl = pl.ds(g * _LANES, _LANES)
            dot = gu_v[0, sl] * gi_v[0, sl]
            for k in range(1, k_dim):
                dot = dot + gu_v[k, sl] * gi_v[k, sl]
            pred = alpha + bu_v[sl] + bi_v[sl] + dot
            sig = 5.0 / (1.0 + jnp.exp(-pred))
            diff = sig - r_v[sl]
            return acc_loss + diff * diff

        acc = lax.fori_loop(0, n_groups, group,
                            jnp.zeros((_LANES,), jnp.float32))
        loss_v[...] = acc
        pltpu.sync_copy(loss_v, out_hbm.at[wid])

    return main_kernel


def kernel(sampleU, sampleI, sampleR, alpha, betaU, betaI, gammaU, gammaI):
    info = plsc.get_sparse_core_info()
    nc, ns = info.num_cores, info.num_subcores
    nw = nc * ns
    b = sampleU.shape[0]
    k_dim = gammaU.shape[1]
    b_per_w = b // nw
    su = sampleU.astype(jnp.int32).reshape(nw, b_per_w)
    si = sampleI.astype(jnp.int32).reshape(nw, b_per_w)
    r = sampleR.astype(jnp.float32).reshape(nw, b_per_w)
    av = jnp.broadcast_to(jnp.asarray(alpha, jnp.float32), (_LANES,))
    gU_flat, nt_u = _flatten_table(gammaU)
    gI_flat, nt_i = _flatten_table(gammaI)
    prep = _make_prep_kernel(nw, nc, b_per_w, k_dim, nt_i)
    bu, bi, giv = prep(su, si, betaU, betaI, gI_flat)
    main = _make_main_kernel(nw, nc, b_per_w, k_dim, nt_u)
    out = main(su, r, av, gU_flat, bu, bi, giv)
    return jnp.sum(out) / b
